# Initial kernel scaffold; baseline (speedup 1.0000x reference)
import jax
import jax.numpy as jnp
from jax.experimental import pallas as pl


def kernel(x, router_w, gate_w, up_w, down_w):
    raise NotImplementedError("write your pallas kernel here")


# speedup vs baseline: 6.8441x; 6.8441x over previous
import functools

import jax
import jax.numpy as jnp
from jax.experimental import pallas as pl
from jax.experimental.pallas import tpu as pltpu

S = 2048
D_MODEL = 768
D_FF = 2048
NUM_EXPERTS = 64
AUX_COEF = 0.01

TM = 256
NT = S // TM
NSTEPS = (NUM_EXPERTS + 1) + (NT - 1) - 1


def _router_body(x_ref, rw_ref, ti_ref, w_ref, ps_ref):
    logits = jnp.dot(x_ref[...], rw_ref[...], preferred_element_type=jnp.float32)
    m = jnp.max(logits, axis=1, keepdims=True)
    ex = jnp.exp(logits - m)
    probs = ex / jnp.sum(ex, axis=1, keepdims=True)
    pmax = jnp.max(probs, axis=1, keepdims=True)
    ii = jax.lax.broadcasted_iota(jnp.int32, (S, NUM_EXPERTS), 1)
    idx = jnp.min(jnp.where(probs == pmax, ii, NUM_EXPERTS), axis=1, keepdims=True)
    ti_ref[...] = idx
    w_ref[...] = pmax / (pmax + 1e-9)
    ps_ref[...] = jnp.sum(probs, axis=0, keepdims=True)


def _router(x_flat, router_w):
    return pl.pallas_call(
        _router_body,
        out_shape=(
            jax.ShapeDtypeStruct((S, 1), jnp.int32),
            jax.ShapeDtypeStruct((S, 1), jnp.float32),
            jax.ShapeDtypeStruct((1, NUM_EXPERTS), jnp.float32),
        ),
    )(x_flat, router_w)


def _ffn_body(meta_ref, xs_ref, ws_ref, gw_ref, uw_ref, dw_ref, out_ref):
    s = pl.program_id(0)
    tile = meta_ref[1, s]
    prev_tile = meta_ref[1, jnp.maximum(s - 1, 0)]
    first = jnp.logical_or(s == 0, tile != prev_tile)
    start = meta_ref[2, s] - tile * TM
    end = meta_ref[3, s] - tile * TM

    @pl.when(first)
    def _():
        out_ref[...] = jnp.zeros_like(out_ref)

    @pl.when(end > start)
    def _():
        xt = xs_ref[...]
        g = jnp.dot(xt, gw_ref[0], preferred_element_type=jnp.float32)
        u = jnp.dot(xt, uw_ref[0], preferred_element_type=jnp.float32)
        h = g * jax.lax.logistic(g) * u
        eo = jnp.dot(h, dw_ref[0], preferred_element_type=jnp.float32)
        rows = jax.lax.broadcasted_iota(jnp.int32, (TM, 1), 0)
        msk = jnp.logical_and(rows >= start, rows < end)
        out_ref[...] += eo * jnp.where(msk, ws_ref[...], 0.0)


def _grouped_ffn(meta, xs, ws_col, gate_w, up_w, down_w):
    grid_spec = pltpu.PrefetchScalarGridSpec(
        num_scalar_prefetch=1,
        grid=(NSTEPS,),
        in_specs=[
            pl.BlockSpec((TM, D_MODEL), lambda s, m: (m[1, s], 0)),
            pl.BlockSpec((TM, 1), lambda s, m: (m[1, s], 0)),
            pl.BlockSpec((1, D_MODEL, D_FF), lambda s, m: (m[0, s], 0, 0)),
            pl.BlockSpec((1, D_MODEL, D_FF), lambda s, m: (m[0, s], 0, 0)),
            pl.BlockSpec((1, D_FF, D_MODEL), lambda s, m: (m[0, s], 0, 0)),
        ],
        out_specs=pl.BlockSpec((TM, D_MODEL), lambda s, m: (m[1, s], 0)),
    )
    return pl.pallas_call(
        _ffn_body,
        grid_spec=grid_spec,
        out_shape=jax.ShapeDtypeStruct((S, D_MODEL), jnp.float32),
        compiler_params=pltpu.CompilerParams(
            dimension_semantics=("arbitrary",),
        ),
    )(meta, xs, ws_col, gate_w, up_w, down_w)


def _schedule(counts):
    offs = jnp.concatenate(
        [jnp.zeros((1,), jnp.int32), jnp.cumsum(counts, dtype=jnp.int32)])
    interior = (jnp.arange(1, NT, dtype=jnp.int32) * TM)
    bounds = jnp.sort(jnp.concatenate([offs, interior]))
    start = bounds[:-1]
    end = bounds[1:]
    expert = jnp.clip(
        jnp.searchsorted(offs, start, side="right").astype(jnp.int32) - 1,
        0, NUM_EXPERTS - 1)
    tile = jnp.clip(start // TM, 0, NT - 1)
    return jnp.stack([expert, tile, start, end]).astype(jnp.int32)


def kernel(x, router_w, gate_w, up_w, down_w):
    x_flat = x.reshape(S, D_MODEL)
    ti, w, psum = _router(x_flat, router_w)
    ti = ti.reshape(S)
    w = w.reshape(S)

    counts = jnp.zeros((NUM_EXPERTS,), jnp.int32).at[ti].add(1)
    meta = _schedule(counts)

    order = jnp.argsort(ti)
    xs = x_flat[order]
    ws_col = w[order].reshape(S, 1)

    out_sorted = _grouped_ffn(meta, xs, ws_col, gate_w, up_w, down_w)

    output = jnp.zeros_like(x_flat).at[order].set(out_sorted)
    output = output.reshape(x.shape)

    psum = psum.reshape(NUM_EXPERTS)
    f_frac = counts.astype(jnp.float32) / S
    p_mean = psum / S
    aux_loss = AUX_COEF * NUM_EXPERTS * jnp.sum(f_frac * p_mean)
    return output, aux_loss
